# baseline (speedup 1.0000x reference)
import functools

import jax
import jax.numpy as jnp
from jax import lax
from jax.experimental import pallas as pl
from jax.experimental.pallas import tpu as pltpu
from jax.experimental.pallas import tpu_sc as plsc

_IDX_LANES = 128


@functools.partial(jax.jit, static_argnames=("num_rows", "dim", "chunk_rows"))
def _sc_gather(idx2d, table, *, num_rows, dim, chunk_rows):
    info = plsc.get_sparse_core_info()
    ncores, nsub = info.num_cores, info.num_subcores
    nw = ncores * nsub
    rows_pw = num_rows // nw
    n_chunks = rows_pw // chunk_rows
    chunk_elems = chunk_rows * _IDX_LANES

    mesh = plsc.VectorSubcoreMesh(core_axis_name="c", subcore_axis_name="s")

    @functools.partial(
        pl.kernel,
        out_type=jax.ShapeDtypeStruct((num_rows * _IDX_LANES, dim), jnp.float32),
        mesh=mesh,
        scratch_types=[
            pltpu.VMEM((chunk_rows, _IDX_LANES), jnp.int32),
            pltpu.VMEM((chunk_elems, dim), jnp.float32),
            pltpu.SemaphoreType.DMA,
        ],
        compiler_params=pltpu.CompilerParams(use_tc_tiling_on_sc=False),
    )
    def k(idx_hbm, table_hbm, out_hbm, idx_v, rows_v, sem):
        wid = lax.axis_index("s") * ncores + lax.axis_index("c")
        row0 = wid * rows_pw

        def chunk(g, carry):
            r0 = row0 + g * chunk_rows
            pltpu.sync_copy(idx_hbm.at[pl.ds(r0, chunk_rows), :], idx_v)
            copies = [
                pltpu.async_copy(
                    table_hbm.at[idx_v.at[j]],
                    rows_v.at[pl.ds(j * _IDX_LANES, _IDX_LANES)],
                    sem,
                )
                for j in range(chunk_rows)
            ]
            for cp in copies:
                cp.wait()
            pltpu.sync_copy(rows_v, out_hbm.at[pl.ds(r0 * _IDX_LANES, chunk_elems)])
            return carry

        lax.fori_loop(0, n_chunks, chunk, 0)

    return k(idx2d, table)


def kernel(token_ids, embedding_matrix):
    b, s = token_ids.shape
    _, dim = embedding_matrix.shape
    total = b * s
    num_rows = total // _IDX_LANES
    idx2d = token_ids.reshape(num_rows, _IDX_LANES).astype(jnp.int32)
    out = _sc_gather(idx2d, embedding_matrix,
                     num_rows=num_rows, dim=dim, chunk_rows=4)
    return out.reshape(b, s, dim)


# speedup vs baseline: 1.0428x; 1.0428x over previous
import functools

import jax
import jax.numpy as jnp
from jax import lax
from jax.experimental import pallas as pl
from jax.experimental.pallas import tpu as pltpu
from jax.experimental.pallas import tpu_sc as plsc

_IDX_LANES = 128
_NSLOTS = 2


@functools.partial(jax.jit, static_argnames=("num_rows", "dim", "chunk_rows"))
def _sc_gather(idx2d, table, *, num_rows, dim, chunk_rows):
    info = plsc.get_sparse_core_info()
    ncores, nsub = info.num_cores, info.num_subcores
    nw = ncores * nsub
    rows_pw = num_rows // nw
    n_chunks = rows_pw // chunk_rows
    n_iters = n_chunks // _NSLOTS
    chunk_elems = chunk_rows * _IDX_LANES

    mesh = plsc.VectorSubcoreMesh(core_axis_name="c", subcore_axis_name="s")

    @functools.partial(
        pl.kernel,
        out_type=jax.ShapeDtypeStruct((num_rows * _IDX_LANES, dim), jnp.float32),
        mesh=mesh,
        scratch_types=[
            pltpu.VMEM((_NSLOTS, chunk_rows, _IDX_LANES), jnp.int32),
            pltpu.VMEM((_NSLOTS * chunk_elems, dim), jnp.float32),
            [pltpu.SemaphoreType.DMA] * _NSLOTS,
            [pltpu.SemaphoreType.DMA] * _NSLOTS,
            [pltpu.SemaphoreType.DMA] * _NSLOTS,
        ],
        compiler_params=pltpu.CompilerParams(use_tc_tiling_on_sc=False),
    )
    def k(idx_hbm, table_hbm, out_hbm, idx_v, rows_v, gsems, osems, isems):
        wid = lax.axis_index("s") * ncores + lax.axis_index("c")
        row0 = wid * rows_pw

        def idx_copy(chunk_id, slot, sem):
            return pltpu.make_async_copy(
                idx_hbm.at[pl.ds(row0 + chunk_id * chunk_rows, chunk_rows), :],
                idx_v.at[slot], sem)

        def wb_copy(chunk_id, slot, sem):
            return pltpu.make_async_copy(
                rows_v.at[pl.ds(slot * chunk_elems, chunk_elems), :],
                out_hbm.at[pl.ds((row0 + chunk_id * chunk_rows) * _IDX_LANES,
                                 chunk_elems), :],
                sem)

        for p in range(_NSLOTS):
            idx_copy(p, p, isems[p]).start()

        def body(i, carry):
            for p in range(_NSLOTS):
                g = i * _NSLOTS + p
                @pl.when(i >= 1)
                def _():
                    wb_copy(g, p, osems[p]).wait()
                idx_copy(g, p, isems[p]).wait()
                copies = [
                    pltpu.async_copy(
                        table_hbm.at[idx_v.at[p, j]],
                        rows_v.at[pl.ds(p * chunk_elems + j * _IDX_LANES,
                                        _IDX_LANES)],
                        gsems[p],
                    )
                    for j in range(chunk_rows)
                ]
                for cp in copies:
                    cp.wait()
                idx_copy(lax.rem(g + _NSLOTS, n_chunks), p, isems[p]).start()
                wb_copy(g, p, osems[p]).start()
            return carry

        lax.fori_loop(0, n_iters, body, 0)

        for p in range(_NSLOTS):
            wb_copy(0, p, osems[p]).wait()
            idx_copy(0, p, isems[p]).wait()

    return k(idx2d, table)


def kernel(token_ids, embedding_matrix):
    b, s = token_ids.shape
    _, dim = embedding_matrix.shape
    total = b * s
    num_rows = total // _IDX_LANES
    idx2d = token_ids.reshape(num_rows, _IDX_LANES).astype(jnp.int32)
    out = _sc_gather(idx2d, embedding_matrix,
                     num_rows=num_rows, dim=dim, chunk_rows=4)
    return out.reshape(b, s, dim)
